# Initial kernel scaffold; baseline (speedup 1.0000x reference)
import jax
import jax.numpy as jnp
from jax.experimental import pallas as pl


def kernel(token_ids, weights):
    raise NotImplementedError("write your pallas kernel here")


# speedup vs baseline: 1.0993x; 1.0993x over previous
import jax
import jax.numpy as jnp
from jax.experimental import pallas as pl
from jax.experimental.pallas import tpu as pltpu
from jax.experimental.pallas import tpu_sc as plsc


def kernel(token_ids, weights):
    B, S = token_ids.shape
    N = B * S
    D = weights.shape[1]

    W = 512
    assert N % W == 0

    idx = token_ids.reshape(1, N).astype(jnp.int32)

    mesh = plsc.VectorSubcoreMesh(
        core_axis_name="core", subcore_axis_name="subcore"
    )

    @jax.jit
    def run(weights, idx):
        @pl.kernel(
            out_type=jax.ShapeDtypeStruct((N, D), weights.dtype),
            mesh=mesh,
            compiler_params=pltpu.CompilerParams(use_tc_tiling_on_sc=False),
        )
        def gather_kernel(w_hbm, i_hbm, o_hbm):
            def body(i_vmem, o_vmem):
                pltpu.sync_copy(w_hbm.at[i_vmem.at[0]], o_vmem)

            pltpu.emit_pipeline(
                body,
                grid=(N // W,),
                in_specs=[
                    pl.BlockSpec((1, W), index_map=lambda i: (0, i))
                ],
                out_specs=[
                    pl.BlockSpec((W, D), index_map=lambda i: (i, 0))
                ],
                core_axis_name=("core", "subcore"),
                dimension_semantics=(pltpu.PARALLEL,),
            )(i_hbm, o_hbm)

        return gather_kernel(weights, idx)

    return run(weights, idx).reshape(B, S, D)


# speedup vs baseline: 1.1103x; 1.0101x over previous
import jax
import jax.numpy as jnp
from jax.experimental import pallas as pl
from jax.experimental.pallas import tpu as pltpu
from jax.experimental.pallas import tpu_sc as plsc


def kernel(token_ids, weights):
    B, S = token_ids.shape
    N = B * S
    D = weights.shape[1]

    W = 1024
    assert N % W == 0

    idx = token_ids.reshape(1, N).astype(jnp.int32)

    mesh = plsc.VectorSubcoreMesh(
        core_axis_name="core", subcore_axis_name="subcore"
    )

    @jax.jit
    def run(weights, idx):
        @pl.kernel(
            out_type=jax.ShapeDtypeStruct((N, D), weights.dtype),
            mesh=mesh,
            compiler_params=pltpu.CompilerParams(use_tc_tiling_on_sc=False),
        )
        def gather_kernel(w_hbm, i_hbm, o_hbm):
            def body(i_vmem, o_vmem):
                pltpu.sync_copy(w_hbm.at[i_vmem.at[0]], o_vmem)

            pltpu.emit_pipeline(
                body,
                grid=(N // W,),
                in_specs=[
                    pl.BlockSpec((1, W), index_map=lambda i: (0, i))
                ],
                out_specs=[
                    pl.BlockSpec((W, D), index_map=lambda i: (i, 0))
                ],
                core_axis_name=("core", "subcore"),
                dimension_semantics=(pltpu.PARALLEL,),
            )(i_hbm, o_hbm)

        return gather_kernel(weights, idx)

    return run(weights, idx).reshape(B, S, D)


# speedup vs baseline: 1.4110x; 1.2708x over previous
import jax
import jax.numpy as jnp
from jax.experimental import pallas as pl
from jax.experimental.pallas import tpu as pltpu
from jax.experimental.pallas import tpu_sc as plsc


def kernel(token_ids, weights):
    B, S = token_ids.shape
    V, D = weights.shape
    L = 16

    W = 512
    assert B % W == 0 and W % L == 0

    idx = token_ids.T.astype(jnp.int32)

    mesh = plsc.VectorSubcoreMesh(
        core_axis_name="core", subcore_axis_name="subcore"
    )

    @jax.jit
    def run(weights, idx):
        @pl.kernel(
            out_type=jax.ShapeDtypeStruct((S, D, B), weights.dtype),
            mesh=mesh,
            scratch_types=[pltpu.VMEM((W, D), weights.dtype)],
            compiler_params=pltpu.CompilerParams(
                use_tc_tiling_on_sc=False, needs_layout_passes=False
            ),
        )
        def gather_kernel(w_hbm, i_hbm, o_hbm, g_ref):
            def body(i_vmem, o_vmem):
                pltpu.sync_copy(w_hbm.at[i_vmem.at[0]], g_ref)

                @pl.loop(0, W // L)
                def _(rc):
                    rows = jax.lax.iota(jnp.int32, L) + rc * L
                    for c in range(D):
                        cols = jnp.full((L,), c, jnp.int32)
                        vals = plsc.load_gather(g_ref, [rows, cols])
                        o_vmem[0, c, pl.ds(rc * L, L)] = vals

            pltpu.emit_pipeline(
                body,
                grid=(S, B // W),
                in_specs=[
                    pl.BlockSpec((1, W), index_map=lambda s, i: (s, i))
                ],
                out_specs=[
                    pl.BlockSpec((1, D, W), index_map=lambda s, i: (s, 0, i))
                ],
                core_axis_name=("core", "subcore"),
                dimension_semantics=(pltpu.PARALLEL, pltpu.PARALLEL),
            )(i_hbm, o_hbm)

        return gather_kernel(weights, idx)

    out = run(weights, idx)
    return jnp.transpose(out, (2, 0, 1))


# speedup vs baseline: 1.9000x; 1.3466x over previous
import jax
import jax.numpy as jnp
from jax.experimental import pallas as pl
from jax.experimental.pallas import tpu as pltpu
from jax.experimental.pallas import tpu_sc as plsc


def kernel(token_ids, weights):
    B, S = token_ids.shape
    V, D = weights.shape

    W = 512
    assert B % W == 0

    idx = token_ids.T.astype(jnp.int32)

    mesh = plsc.VectorSubcoreMesh(
        core_axis_name="core", subcore_axis_name="subcore"
    )

    @jax.jit
    def run(weights, idx):
        @pl.kernel(
            out_type=jax.ShapeDtypeStruct((S, B, D), weights.dtype),
            mesh=mesh,
            compiler_params=pltpu.CompilerParams(use_tc_tiling_on_sc=False),
        )
        def gather_kernel(w_hbm, i_hbm, o_hbm):
            def body(i_vmem, o_vmem):
                pltpu.sync_copy(w_hbm.at[i_vmem.at[0]], o_vmem.at[0])

            pltpu.emit_pipeline(
                body,
                grid=(S, B // W),
                in_specs=[
                    pl.BlockSpec((1, W), index_map=lambda s, i: (s, i))
                ],
                out_specs=[
                    pl.BlockSpec((1, W, D), index_map=lambda s, i: (s, i, 0))
                ],
                core_axis_name=("core", "subcore"),
                dimension_semantics=(pltpu.PARALLEL, pltpu.PARALLEL),
            )(i_hbm, o_hbm)

        return gather_kernel(weights, idx)

    out = run(weights, idx)
    return jnp.transpose(out, (1, 0, 2))


# speedup vs baseline: 1.9216x; 1.0114x over previous
import jax
import jax.numpy as jnp
from jax.experimental import pallas as pl
from jax.experimental.pallas import tpu as pltpu
from jax.experimental.pallas import tpu_sc as plsc


def kernel(token_ids, weights):
    B, S = token_ids.shape
    V, D = weights.shape

    W = 512
    K = 2048
    assert B % W == 0 and K % 4 == 0

    idx = token_ids.T.astype(jnp.int32)
    wT = weights.T

    mesh = plsc.VectorSubcoreMesh(
        core_axis_name="core", subcore_axis_name="subcore"
    )

    @jax.jit
    def run(wT, idx):
        def pack_body(in_ref, out_ref):
            x = in_ref[...]
            y = jnp.transpose(x)
            y3 = y.reshape(K // 4, 4, D)
            out_ref[...] = jnp.concatenate(
                [y3[:, j, :] for j in range(4)], axis=1
            )

        packed = pl.pallas_call(
            pack_body,
            grid=((V + K - 1) // K,),
            in_specs=[pl.BlockSpec((D, K), lambda i: (0, i))],
            out_specs=pl.BlockSpec((K // 4, 4 * D), lambda i: (i, 0)),
            out_shape=jax.ShapeDtypeStruct((V // 4, 4 * D), wT.dtype),
        )(wT)
        table = packed.reshape(V, D)

        @pl.kernel(
            out_type=jax.ShapeDtypeStruct((S, B, D), wT.dtype),
            mesh=mesh,
            compiler_params=pltpu.CompilerParams(use_tc_tiling_on_sc=False),
        )
        def gather_kernel(w_hbm, i_hbm, o_hbm):
            def body(i_vmem, o_vmem):
                pltpu.sync_copy(w_hbm.at[i_vmem.at[0]], o_vmem.at[0])

            pltpu.emit_pipeline(
                body,
                grid=(S, B // W),
                in_specs=[
                    pl.BlockSpec((1, W), index_map=lambda s, i: (s, i))
                ],
                out_specs=[
                    pl.BlockSpec((1, W, D), index_map=lambda s, i: (s, i, 0))
                ],
                core_axis_name=("core", "subcore"),
                dimension_semantics=(pltpu.PARALLEL, pltpu.PARALLEL),
            )(i_hbm, o_hbm)

        return gather_kernel(table, idx)

    out = run(wT, idx)
    return jnp.transpose(out, (1, 0, 2))
